# Initial kernel scaffold; baseline (speedup 1.0000x reference)
import jax
import jax.numpy as jnp
from jax.experimental import pallas as pl


def kernel(x, edge_attr, W_self, W_nbr, W_edge, b, edge_index, mask):
    raise NotImplementedError("write your pallas kernel here")


# speedup vs baseline: 11.5861x; 11.5861x over previous
import functools

import jax
import jax.numpy as jnp
from jax import lax
from jax.experimental import pallas as pl
from jax.experimental.pallas import tpu as pltpu
from jax.experimental.pallas import tpu_sc as plsc

N_NODES = 10000
N_EDGES = 320000
D_FEAT = 128
D_EDGE = 4
D_EDGE_PAD = 16

NC = 2
NS = 16
NW = NC * NS

ROWS_PAD = 10240
ROWS_PER_TILE = ROWS_PAD // NS
DUMMY_ROW = 10200
EDGES_PER_WORKER = N_EDGES // NW
MEGA = 400
N_MEGA = EDGES_PER_WORKER // MEGA
SUB = 80
N_SUB = MEGA // SUB
VECS_PER_MEGA = MEGA // 16


def _make_sc_kernel():
    mesh = plsc.VectorSubcoreMesh(core_axis_name="c", subcore_axis_name="s")

    @functools.partial(
        pl.kernel,
        out_type=[
            jax.ShapeDtypeStruct((NC, ROWS_PAD, D_FEAT), jnp.float32),
            jax.ShapeDtypeStruct((NC, ROWS_PAD, D_EDGE_PAD), jnp.float32),
        ],
        mesh=mesh,
        compiler_params=pltpu.CompilerParams(
            needs_layout_passes=False, use_tc_tiling_on_sc=False),
        scratch_types=[
            pltpu.VMEM((N_NODES,), jnp.int32),
            pltpu.VMEM((MEGA,), jnp.int32),
            pltpu.VMEM((MEGA,), jnp.int32),
            pltpu.VMEM((MEGA, D_EDGE_PAD), jnp.float32),
            pltpu.VMEM((N_SUB, SUB), jnp.int32),
            pltpu.VMEM((SUB, D_FEAT), jnp.float32),
            pltpu.VMEM_SHARED((ROWS_PAD, D_FEAT), jnp.float32),
            pltpu.VMEM_SHARED((ROWS_PAD, D_EDGE_PAD), jnp.float32),
        ],
    )
    def sc_kernel(x_hbm, src_hbm, dst_hbm, ea_hbm, mask_hbm, zg_hbm, ze_hbm,
                  g_out, e_out,
                  mask_v, src_v, dst_v, ea_v, eff2, rows_v, g_sh, e_sh):
        cid = lax.axis_index("c")
        sid = lax.axis_index("s")
        wid = cid * NS + sid
        r0 = sid * ROWS_PER_TILE

        pltpu.sync_copy(zg_hbm.at[pl.ds(r0, ROWS_PER_TILE)],
                        g_sh.at[pl.ds(r0, ROWS_PER_TILE)])
        pltpu.sync_copy(ze_hbm.at[pl.ds(r0, ROWS_PER_TILE)],
                        e_sh.at[pl.ds(r0, ROWS_PER_TILE)])

        pltpu.sync_copy(mask_hbm, mask_v)

        plsc.subcore_barrier()

        base_w = wid * EDGES_PER_WORKER

        for m in range(N_MEGA):
            base = base_w + m * MEGA
            pltpu.sync_copy(src_hbm.at[pl.ds(base, MEGA)], src_v)
            pltpu.sync_copy(dst_hbm.at[pl.ds(base, MEGA)], dst_v)
            pltpu.sync_copy(ea_hbm.at[pl.ds(base, MEGA)], ea_v)

            def eff_body(i, _):
                sv = src_v[pl.ds(i * 16, 16)]
                dv = dst_v[pl.ds(i * 16, 16)]
                ms = plsc.load_gather(mask_v, [sv])
                md = plsc.load_gather(mask_v, [dv])
                keep = (ms & md) > 0
                eff2[i // 5, pl.ds((i % 5) * 16, 16)] = jnp.where(keep, dv, DUMMY_ROW)
                return 0
            lax.fori_loop(0, VECS_PER_MEGA, eff_body, 0)

            def dma_body(k, _):
                pltpu.sync_copy(x_hbm.at[src_v.at[pl.ds(k * SUB, SUB)]], rows_v)
                pltpu.sync_copy(rows_v, g_sh.at[eff2.at[k]], add=True)
                pltpu.sync_copy(ea_v.at[pl.ds(k * SUB, SUB)],
                                e_sh.at[eff2.at[k]], add=True)
                return 0
            lax.fori_loop(0, N_SUB, dma_body, 0)

        plsc.subcore_barrier()

        pltpu.sync_copy(g_sh.at[pl.ds(r0, ROWS_PER_TILE)],
                        g_out.at[cid, pl.ds(r0, ROWS_PER_TILE)])
        pltpu.sync_copy(e_sh.at[pl.ds(r0, ROWS_PER_TILE)],
                        e_out.at[cid, pl.ds(r0, ROWS_PER_TILE)])

    return sc_kernel


def _dense_body(x_ref, gp_ref, ep_ref, ws_ref, wn_ref, we_ref, b_ref, m_ref, o_ref):
    g = gp_ref[0] + gp_ref[1]
    e = ep_ref[0] + ep_ref[1]
    acc = jnp.dot(x_ref[...], ws_ref[...], preferred_element_type=jnp.float32)
    acc = acc + jnp.dot(g, wn_ref[...], preferred_element_type=jnp.float32)
    acc = acc + jnp.dot(e, we_ref[...], preferred_element_type=jnp.float32)
    acc = acc + b_ref[...]
    o_ref[...] = jnp.maximum(acc, 0.0) * m_ref[...]


_R = 400


def _dense_call(x, gp, ep, W_self, W_nbr, W_edge, b2, m2):
    return pl.pallas_call(
        _dense_body,
        grid=(N_NODES // _R,),
        in_specs=[
            pl.BlockSpec((_R, D_FEAT), lambda i: (i, 0)),
            pl.BlockSpec((NC, _R, D_FEAT), lambda i: (0, i, 0)),
            pl.BlockSpec((NC, _R, D_EDGE_PAD), lambda i: (0, i, 0)),
            pl.BlockSpec((D_FEAT, D_FEAT), lambda i: (0, 0)),
            pl.BlockSpec((D_FEAT, D_FEAT), lambda i: (0, 0)),
            pl.BlockSpec((D_EDGE_PAD, D_FEAT), lambda i: (0, 0)),
            pl.BlockSpec((1, D_FEAT), lambda i: (0, 0)),
            pl.BlockSpec((_R, 1), lambda i: (i, 0)),
        ],
        out_specs=pl.BlockSpec((_R, D_FEAT), lambda i: (i, 0)),
        out_shape=jax.ShapeDtypeStruct((N_NODES, D_FEAT), jnp.float32),
    )(x, gp, ep, W_self, W_nbr, W_edge, b2, m2)


def kernel(x, edge_attr, W_self, W_nbr, W_edge, b, edge_index, mask):
    mask_i32 = mask.astype(jnp.int32)
    zg = jnp.zeros((ROWS_PAD, D_FEAT), jnp.float32)
    ze = jnp.zeros((ROWS_PAD, D_EDGE_PAD), jnp.float32)
    sc = _make_sc_kernel()
    ea_pad = jnp.pad(edge_attr, ((0, 0), (0, D_EDGE_PAD - D_EDGE)))
    gp, ep = sc(x, edge_index[0], edge_index[1], ea_pad, mask_i32, zg, ze)
    b2 = b.reshape(1, D_FEAT)
    m2 = mask.astype(jnp.float32).reshape(N_NODES, 1)
    we_pad = jnp.pad(W_edge, ((0, D_EDGE_PAD - D_EDGE), (0, 0)))
    return _dense_call(x, gp, ep, W_self, W_nbr, we_pad, b2, m2)
